# baseline (speedup 1.0000x reference)
import functools

import jax
import jax.numpy as jnp
import numpy as np
from jax import lax
from jax.experimental import pallas as pl
from jax.experimental.pallas import tpu as pltpu
from jax.experimental.pallas import tpu_sc as plsc

VOCAB = 32000
D_A = 768
D_B = 768
N_HEADS = 12
D_H = 64
D_FF = 3072
N_POOL = 8192
R = 2
TOP_K = 8
D_K = 64
N_ASPECTS = 4
T = 2048
D_POOL = R * (D_A + D_B)

TT = 256
NT = T // TT
PT = 1024
NPT = N_POOL // PT
VT = 1280
NVT = VOCAB // VT

_F32 = jnp.float32


def _ln_in(x, s, b):
    m = jnp.mean(x, axis=-1, keepdims=True)
    v = jnp.mean((x - m) ** 2, axis=-1, keepdims=True)
    return (x - m) * lax.rsqrt(v + 1e-5) * s + b


def _pos_enc_const(seq_len, d_model):
    pos = np.arange(seq_len, dtype=np.float32)[:, None]
    i = np.arange(d_model // 2, dtype=np.float32)[None, :]
    angle = (pos / (10000.0 ** (2.0 * i / d_model))).astype(np.float32)
    enc = np.concatenate([np.sin(angle), np.cos(angle)], axis=-1)
    return enc[:, :d_model].astype(np.float32)


_POS_ENC = _pos_enc_const(T, D_A)


def _embed_gather(table, idx):
    info = plsc.get_sparse_core_info()
    nc, ns = info.num_cores, info.num_subcores
    nw = nc * ns
    bpw = T // nw
    mesh = plsc.VectorSubcoreMesh(core_axis_name="c", subcore_axis_name="s")

    @functools.partial(
        pl.kernel, mesh=mesh,
        out_type=jax.ShapeDtypeStruct((T, D_A), _F32),
        scratch_types=[
            pltpu.VMEM((bpw,), jnp.int32),
            pltpu.VMEM((bpw, D_A), _F32),
            pltpu.SemaphoreType.DMA,
        ],
    )
    def k(table_hbm, idx_hbm, out_hbm, idx_v, rows_v, sem):
        wid = lax.axis_index("s") * nc + lax.axis_index("c")
        base = wid * bpw
        pltpu.sync_copy(idx_hbm.at[pl.ds(base, bpw)], idx_v)
        pltpu.async_copy(table_hbm.at[idx_v], rows_v, sem).wait()
        pltpu.sync_copy(rows_v, out_hbm.at[pl.ds(base, bpw)])

    return k(table, idx)


def _pool_gather(pool, idx):
    info = plsc.get_sparse_core_info()
    nc = info.num_cores
    mesh = plsc.VectorSubcoreMesh(core_axis_name="c", subcore_axis_name="s")

    @functools.partial(
        pl.kernel, mesh=mesh,
        out_type=jax.ShapeDtypeStruct((TOP_K, D_POOL), _F32),
        scratch_types=[
            pltpu.VMEM((TOP_K,), jnp.int32),
            pltpu.VMEM((TOP_K, D_POOL), _F32),
            pltpu.SemaphoreType.DMA,
        ],
    )
    def k(pool_hbm, idx_hbm, out_hbm, idx_v, rows_v, sem):
        wid = lax.axis_index("s") * nc + lax.axis_index("c")

        @pl.when(wid == 0)
        def _():
            pltpu.sync_copy(idx_hbm, idx_v)
            pltpu.async_copy(pool_hbm.at[idx_v], rows_v, sem).wait()
            pltpu.sync_copy(rows_v, out_hbm)

    return k(pool, idx)


DP = 128
DAP = N_HEADS * DP


def _pad_heads(qkv, off):
    pieces = []
    for h in range(N_HEADS):
        pieces.append(qkv[:, off + h * D_H:off + (h + 1) * D_H])
        pieces.append(jnp.zeros((qkv.shape[0], DP - D_H), _F32))
    return jnp.concatenate(pieces, axis=1)


def _attn_inner(i, qkv, k_scr, v_scr, o_ref):
    def attn_len(L):
        row = i * TT + lax.broadcasted_iota(jnp.int32, (TT, L), 0)
        col = lax.broadcasted_iota(jnp.int32, (TT, L), 1)
        madd = jnp.where(col <= row, _F32(0.0), _F32(-1e9))
        outs = []
        for h in range(N_HEADS):
            qh = qkv[:, h * D_H:(h + 1) * D_H] * 0.125
            kh = k_scr[0:L, h * DP:(h + 1) * DP]
            vh = v_scr[0:L, h * DP:(h + 1) * DP]
            s = lax.dot_general(
                jnp.concatenate(
                    [qh, jnp.zeros((TT, DP - D_H), _F32)], axis=1),
                kh, (((1,), (1,)), ((), ())),
                preferred_element_type=_F32) + madd
            m = jnp.max(s, axis=-1, keepdims=True)
            e = jnp.exp(s - m)
            rden = 1.0 / jnp.sum(e, axis=-1, keepdims=True)
            outs.append(jnp.dot(e, vh,
                                preferred_element_type=_F32)[:, :D_H] * rden)
        o_ref[...] = jnp.concatenate(outs, axis=1)

    for pi in range(NT):

        @pl.when(i == pi)
        def _(pi=pi):
            attn_len((pi + 1) * TT)


def _qkvattn_a_call(g, pos, s1, b1, wqkv):
    def body(g_ref, p_ref, s_ref, b_ref, w_ref, a_ref, x_ref, k_scr, v_scr):
        i = pl.program_id(0)
        x = g_ref[...] + p_ref[...]
        x_ref[...] = x
        h = _ln_in(x, s_ref[...], b_ref[...])
        qkv = jnp.dot(h, w_ref[...], preferred_element_type=_F32)
        k_scr[pl.ds(i * TT, TT), :] = _pad_heads(qkv, D_A)
        v_scr[pl.ds(i * TT, TT), :] = _pad_heads(qkv, 2 * D_A)

        _attn_inner(i, qkv, k_scr, v_scr, a_ref)

    tile = lambda i: (i, 0)
    full = lambda i: (0, 0)
    return pl.pallas_call(
        body,
        grid=(NT,),
        in_specs=[
            pl.BlockSpec((TT, D_A), tile),
            pl.BlockSpec((TT, D_A), tile),
            pl.BlockSpec((1, D_A), full),
            pl.BlockSpec((1, D_A), full),
            pl.BlockSpec((D_A, 3 * D_A), full),
        ],
        out_specs=[
            pl.BlockSpec((TT, D_A), tile),
            pl.BlockSpec((TT, D_A), tile),
        ],
        out_shape=[jax.ShapeDtypeStruct((T, D_A), _F32)] * 2,
        scratch_shapes=[
            pltpu.VMEM((T, DAP), _F32),
            pltpu.VMEM((T, DAP), _F32),
        ],
        compiler_params=pltpu.CompilerParams(
            vmem_limit_bytes=63 * 1024 * 1024),
    )(g, pos, s1, b1, wqkv)


def _qkvattn_b_call(h_a, au, bv, alpha16, w_base, gamma, bb, asm_s, asm_b,
                    s1, b1, wqkv):
    def body(x_ref, au_ref, bv_ref, al_ref, wb_ref, g_ref, bb_ref, as_ref,
             ab_ref, s_ref, b_ref, w_ref, a_ref, hm_ref, k_scr, v_scr,
             wm_scr):
        i = pl.program_id(0)

        @pl.when(i == 0)
        def _():
            delta = jnp.dot(au_ref[...] * al_ref[...], bv_ref[...],
                            preferred_element_type=_F32)
            wm_scr[...] = wb_ref[...] + g_ref[0] * delta

        t = lax.dot_general(x_ref[...], wm_scr[...],
                            (((1,), (1,)), ((), ())),
                            preferred_element_type=_F32) + bb_ref[...]
        hm = _ln_in(t, as_ref[...], ab_ref[...])
        hm_ref[...] = hm
        h = _ln_in(hm, s_ref[...], b_ref[...])
        qkv = jnp.dot(h, w_ref[...], preferred_element_type=_F32)
        k_scr[pl.ds(i * TT, TT), :] = _pad_heads(qkv, D_A)
        v_scr[pl.ds(i * TT, TT), :] = _pad_heads(qkv, 2 * D_A)

        _attn_inner(i, qkv, k_scr, v_scr, a_ref)

    tile = lambda i: (i, 0)
    full = lambda i: (0, 0)
    return pl.pallas_call(
        body,
        grid=(NT,),
        in_specs=[
            pl.BlockSpec((TT, D_A), tile),
            pl.BlockSpec((D_B, 2 * TOP_K), full),
            pl.BlockSpec((2 * TOP_K, D_A), full),
            pl.BlockSpec((1, 2 * TOP_K), full),
            pl.BlockSpec((D_B, D_A), full),
            pl.BlockSpec(memory_space=pltpu.SMEM),
            pl.BlockSpec((1, D_B), full),
            pl.BlockSpec((1, D_B), full),
            pl.BlockSpec((1, D_B), full),
            pl.BlockSpec((1, D_A), full),
            pl.BlockSpec((1, D_A), full),
            pl.BlockSpec((D_A, 3 * D_A), full),
        ],
        out_specs=[
            pl.BlockSpec((TT, D_A), tile),
            pl.BlockSpec((TT, D_B), tile),
        ],
        out_shape=[jax.ShapeDtypeStruct((T, D_A), _F32),
                   jax.ShapeDtypeStruct((T, D_B), _F32)],
        scratch_shapes=[
            pltpu.VMEM((T, DAP), _F32),
            pltpu.VMEM((T, DAP), _F32),
            pltpu.VMEM((D_B, D_A), _F32),
        ],
        compiler_params=pltpu.CompilerParams(
            vmem_limit_bytes=63 * 1024 * 1024),
    )(h_a, au, bv, alpha16, w_base, gamma, bb, asm_s, asm_b, s1, b1, wqkv)


def _woffn_a_call(x, attn, wo, s2, b2, w1, bb1, w2, bb2, wq2, wk2):
    def body(x_ref, a_ref, wo_ref, s_ref, b_ref, w1_ref, b1_ref,
             w2_ref, b2_ref, wq_ref, wk_ref, y_ref, m_ref, z_scr):
        i = pl.program_id(0)
        x1 = x_ref[...] + jnp.dot(a_ref[...], wo_ref[...],
                                  preferred_element_type=_F32)
        h2 = _ln_in(x1, s_ref[...], b_ref[...])
        ff = jax.nn.gelu(jnp.dot(h2, w1_ref[...],
                                 preferred_element_type=_F32) + b1_ref[...])
        y = x1 + jnp.dot(ff, w2_ref[...],
                         preferred_element_type=_F32) + b2_ref[...]
        y_ref[...] = y
        zp = jnp.sum(y, axis=0, keepdims=True)

        @pl.when(i == 0)
        def _():
            z_scr[...] = zp

        @pl.when(i > 0)
        def _():
            z_scr[...] = z_scr[...] + zp

        @pl.when(i == NT - 1)
        def _():
            z = z_scr[...] * (1.0 / T)
            qf = jnp.dot(z, wq_ref[...], preferred_element_type=_F32)
            m = lax.dot_general(qf, wk_ref[...], (((1,), (1,)), ((), ())),
                                preferred_element_type=_F32)
            m_ref[...] = m * 0.125

    tile = lambda i: (i, 0)
    full = lambda i: (0, 0)
    return pl.pallas_call(
        body,
        grid=(NT,),
        in_specs=[
            pl.BlockSpec((TT, D_A), tile),
            pl.BlockSpec((TT, D_A), tile),
            pl.BlockSpec((D_A, D_A), full),
            pl.BlockSpec((1, D_A), full),
            pl.BlockSpec((1, D_A), full),
            pl.BlockSpec((D_A, D_FF), full),
            pl.BlockSpec((1, D_FF), full),
            pl.BlockSpec((D_FF, D_A), full),
            pl.BlockSpec((1, D_A), full),
            pl.BlockSpec((D_A, N_ASPECTS * D_K), full),
            pl.BlockSpec((D_POOL, N_ASPECTS * D_K), full),
        ],
        out_specs=[
            pl.BlockSpec((TT, D_A), tile),
            pl.BlockSpec((1, D_POOL), full),
        ],
        out_shape=[jax.ShapeDtypeStruct((T, D_A), _F32),
                   jax.ShapeDtypeStruct((1, D_POOL), _F32)],
        scratch_shapes=[pltpu.VMEM((1, D_A), _F32)],
    )(x, attn, wo, s2, b2, w1, bb1, w2, bb2, wq2, wk2)


def _woffn_b_call(x, attn, wo, s2, b2, w1, bb1, w2, bb2):
    def body(x_ref, a_ref, wo_ref, s_ref, b_ref, w1_ref, b1_ref,
             w2_ref, b2_ref, y_ref):
        x1 = x_ref[...] + jnp.dot(a_ref[...], wo_ref[...],
                                  preferred_element_type=_F32)
        h2 = _ln_in(x1, s_ref[...], b_ref[...])
        ff = jax.nn.gelu(jnp.dot(h2, w1_ref[...],
                                 preferred_element_type=_F32) + b1_ref[...])
        y_ref[...] = x1 + jnp.dot(ff, w2_ref[...],
                                  preferred_element_type=_F32) + b2_ref[...]

    tile = lambda i: (i, 0)
    full = lambda i: (0, 0)
    return pl.pallas_call(
        body,
        grid=(NT,),
        in_specs=[
            pl.BlockSpec((TT, D_A), tile),
            pl.BlockSpec((TT, D_A), tile),
            pl.BlockSpec((D_A, D_A), full),
            pl.BlockSpec((1, D_A), full),
            pl.BlockSpec((1, D_A), full),
            pl.BlockSpec((D_A, D_FF), full),
            pl.BlockSpec((1, D_FF), full),
            pl.BlockSpec((D_FF, D_A), full),
            pl.BlockSpec((1, D_A), full),
        ],
        out_specs=pl.BlockSpec((TT, D_A), tile),
        out_shape=jax.ShapeDtypeStruct((T, D_A), _F32),
    )(x, attn, wo, s2, b2, w1, bb1, w2, bb2)


def _score_topk_call(pool, m, lam, warm):
    def body(p_ref, m_ref, lam_ref, warm_ref, a_ref, i_ref, c_scr):
        i = pl.program_id(0)
        c_scr[pl.ds(i, 1), :] = lax.dot_general(
            m_ref[...], p_ref[...], (((1,), (1,)), ((), ())),
            preferred_element_type=_F32)

        @pl.when(i == NPT - 1)
        def _():
            c = c_scr[...] * lam_ref[0]
            cmax = jnp.max(c)
            e = jnp.exp(c - cmax)
            soft = e / jnp.sum(e)
            flat = (lax.broadcasted_iota(jnp.int32, (NPT, PT), 0) * PT
                    + lax.broadcasted_iota(jnp.int32, (NPT, PT), 1))
            cur = soft
            vals = []
            for kk in range(TOP_K):
                mx = jnp.max(cur)
                am = jnp.min(jnp.where(cur == mx, flat, jnp.int32(N_POOL)))
                vals.append(mx)
                i_ref[kk] = am
                cur = jnp.where(flat == am, _F32(-1.0), cur)
            vsum = vals[0]
            for kk in range(1, TOP_K):
                vsum = vsum + vals[kk]
            warmb = warm_ref[0] != 0
            for kk in range(TOP_K):
                a_ref[kk] = jnp.where(warmb, vals[kk],
                                      vals[kk] / (vsum + 1e-9))

    return pl.pallas_call(
        body,
        grid=(NPT,),
        in_specs=[
            pl.BlockSpec((PT, D_POOL), lambda i: (i, 0)),
            pl.BlockSpec((1, D_POOL), lambda i: (0, 0)),
            pl.BlockSpec(memory_space=pltpu.SMEM),
            pl.BlockSpec(memory_space=pltpu.SMEM),
        ],
        out_specs=[
            pl.BlockSpec(memory_space=pltpu.SMEM),
            pl.BlockSpec(memory_space=pltpu.SMEM),
        ],
        out_shape=[
            jax.ShapeDtypeStruct((TOP_K,), _F32),
            jax.ShapeDtypeStruct((TOP_K,), jnp.int32),
        ],
        scratch_shapes=[pltpu.VMEM((NPT, PT), _F32)],
    )(pool, m, lam, warm)


def _lmhead_call(x, w):
    def body(x_ref, w_ref, o_ref):
        o_ref[...] = jnp.dot(x_ref[...], w_ref[...],
                             preferred_element_type=_F32)

    return pl.pallas_call(
        body,
        grid=(NVT,),
        in_specs=[
            pl.BlockSpec((T, D_B), lambda j: (0, 0)),
            pl.BlockSpec((D_B, VT), lambda j: (0, j)),
        ],
        out_specs=pl.BlockSpec((T, VT), lambda j: (0, j)),
        out_shape=jax.ShapeDtypeStruct((T, VOCAB), _F32),
    )(x, w)


def kernel(input_ids, lambda_val, is_warmup, embed_table, a_ln1_s, a_ln1_b,
           a_wqkv, a_wo, a_ln2_s, a_ln2_b, a_w1, a_b1, a_w2, a_b2,
           pool_vectors, w_key, w_query, w_base, b_base, gamma, asm_ln_s,
           asm_ln_b, b_ln1_s, b_ln1_b, b_wqkv, b_wo, b_ln2_s, b_ln2_b,
           b_w1, b_b1, b_w2, b_b2, lm_head_w):
    row2 = lambda a: jnp.asarray(a, _F32).reshape(1, -1)

    ids = input_ids.reshape(T).astype(jnp.int32)
    g = _embed_gather(embed_table, ids)
    pos = jnp.asarray(_POS_ENC)

    attn, x = _qkvattn_a_call(g, pos, row2(a_ln1_s), row2(a_ln1_b), a_wqkv)
    wq2 = w_query.transpose(1, 0, 2).reshape(D_A, N_ASPECTS * D_K)
    wk2 = w_key.transpose(1, 0, 2).reshape(D_POOL, N_ASPECTS * D_K)
    h_a, m = _woffn_a_call(x, attn, a_wo, row2(a_ln2_s), row2(a_ln2_b),
                           a_w1, row2(a_b1), a_w2, row2(a_b2), wq2, wk2)

    lam = jnp.asarray(lambda_val, _F32).reshape(1)
    warm = jnp.asarray(is_warmup, jnp.int32).reshape(1)
    alphas, indices = _score_topk_call(pool_vectors, m, lam, warm)

    gathered = _pool_gather(pool_vectors, indices)
    au = gathered[:, :D_B * R].reshape(TOP_K, D_B, R).transpose(1, 0, 2)
    au = au.reshape(D_B, TOP_K * R)
    bv = gathered[:, D_B * R:].reshape(TOP_K * R, D_A)
    alpha16 = jnp.repeat(alphas, R).reshape(1, TOP_K * R)
    attn2, h_mid = _qkvattn_b_call(h_a, au, bv, alpha16, w_base,
                                   gamma.reshape(1), row2(b_base),
                                   row2(asm_ln_s), row2(asm_ln_b),
                                   row2(b_ln1_s), row2(b_ln1_b), b_wqkv)
    h_out = _woffn_b_call(h_mid, attn2, b_wo, row2(b_ln2_s), row2(b_ln2_b),
                          b_w1, row2(b_b1), b_w2, row2(b_b2))

    logits = _lmhead_call(h_out, lm_head_w)
    return logits.reshape(1, T, VOCAB)


# speedup vs baseline: 2.2123x; 2.2123x over previous
import functools

import jax
import jax.numpy as jnp
import numpy as np
from jax import lax
from jax.experimental import pallas as pl
from jax.experimental.pallas import tpu as pltpu
from jax.experimental.pallas import tpu_sc as plsc

VOCAB = 32000
D_A = 768
D_B = 768
N_HEADS = 12
D_H = 64
D_FF = 3072
N_POOL = 8192
R = 2
TOP_K = 8
D_K = 64
N_ASPECTS = 4
T = 2048
D_POOL = R * (D_A + D_B)

TT = 256
NT = T // TT
PT = 1024
NPT = N_POOL // PT
VT = 1280
NVT = VOCAB // VT

_F32 = jnp.float32


def _ln_in(x, s, b):
    m = jnp.mean(x, axis=-1, keepdims=True)
    v = jnp.mean((x - m) ** 2, axis=-1, keepdims=True)
    return (x - m) * lax.rsqrt(v + 1e-5) * s + b


def _pos_enc_const(seq_len, d_model):
    pos = np.arange(seq_len, dtype=np.float32)[:, None]
    i = np.arange(d_model // 2, dtype=np.float32)[None, :]
    angle = (pos / (10000.0 ** (2.0 * i / d_model))).astype(np.float32)
    enc = np.concatenate([np.sin(angle), np.cos(angle)], axis=-1)
    return enc[:, :d_model].astype(np.float32)


_POS_ENC = _pos_enc_const(T, D_A)


def _embed_gather(table, idx):
    info = plsc.get_sparse_core_info()
    nc, ns = info.num_cores, info.num_subcores
    nw = nc * ns
    bpw = T // nw
    mesh = plsc.VectorSubcoreMesh(core_axis_name="c", subcore_axis_name="s")

    @functools.partial(
        pl.kernel, mesh=mesh,
        out_type=jax.ShapeDtypeStruct((T, D_A), _F32),
        scratch_types=[
            pltpu.VMEM((bpw,), jnp.int32),
            pltpu.VMEM((bpw, D_A), _F32),
            pltpu.SemaphoreType.DMA,
        ],
    )
    def k(table_hbm, idx_hbm, out_hbm, idx_v, rows_v, sem):
        wid = lax.axis_index("s") * nc + lax.axis_index("c")
        base = wid * bpw
        pltpu.sync_copy(idx_hbm.at[pl.ds(base, bpw)], idx_v)
        pltpu.async_copy(table_hbm.at[idx_v], rows_v, sem).wait()
        pltpu.sync_copy(rows_v, out_hbm.at[pl.ds(base, bpw)])

    return k(table, idx)


def _pool_gather(pool, idx):
    info = plsc.get_sparse_core_info()
    nc = info.num_cores
    mesh = plsc.VectorSubcoreMesh(core_axis_name="c", subcore_axis_name="s")

    @functools.partial(
        pl.kernel, mesh=mesh,
        out_type=jax.ShapeDtypeStruct((TOP_K, D_POOL), _F32),
        scratch_types=[
            pltpu.VMEM((TOP_K,), jnp.int32),
            pltpu.VMEM((TOP_K, D_POOL), _F32),
            pltpu.SemaphoreType.DMA,
        ],
    )
    def k(pool_hbm, idx_hbm, out_hbm, idx_v, rows_v, sem):
        wid = lax.axis_index("s") * nc + lax.axis_index("c")

        @pl.when(wid == 0)
        def _():
            pltpu.sync_copy(idx_hbm, idx_v)
            pltpu.async_copy(pool_hbm.at[idx_v], rows_v, sem).wait()
            pltpu.sync_copy(rows_v, out_hbm)

    return k(pool, idx)


DP = 128
DAP = N_HEADS * DP


def _pad_heads(qkv, off):
    pieces = []
    for h in range(N_HEADS):
        pieces.append(qkv[:, off + h * D_H:off + (h + 1) * D_H])
        pieces.append(jnp.zeros((qkv.shape[0], DP - D_H), _F32))
    return jnp.concatenate(pieces, axis=1)


def _attn_inner(i, qkv, k_scr, v_scr, o_ref):
    def attn_len(L):
        row = i * TT + lax.broadcasted_iota(jnp.int32, (TT, L), 0)
        col = lax.broadcasted_iota(jnp.int32, (TT, L), 1)
        madd = jnp.where(col <= row, _F32(0.0), _F32(-1e9))
        outs = []
        for h in range(N_HEADS):
            qh = qkv[:, h * D_H:(h + 1) * D_H] * 0.125
            kh = k_scr[0:L, h * DP:(h + 1) * DP]
            vh = v_scr[0:L, h * DP:(h + 1) * DP]
            s = lax.dot_general(
                jnp.concatenate(
                    [qh, jnp.zeros((TT, DP - D_H), _F32)], axis=1),
                kh, (((1,), (1,)), ((), ())),
                preferred_element_type=_F32) + madd
            m = jnp.max(s, axis=-1, keepdims=True)
            e = jnp.exp(s - m)
            rden = 1.0 / jnp.sum(e, axis=-1, keepdims=True)
            outs.append(jnp.dot(e, vh,
                                preferred_element_type=_F32)[:, :D_H] * rden)
        o_ref[...] = jnp.concatenate(outs, axis=1)

    for pi in range(NT // 2):

        @pl.when(i // 2 == pi)
        def _(pi=pi):
            attn_len((pi + 1) * 2 * TT)


def _qkvattn_a_call(g, pos, s1, b1, wqkv):
    def body(g_ref, p_ref, s_ref, b_ref, w_ref, a_ref, x_ref, k_scr, v_scr):
        i = pl.program_id(0)
        x = g_ref[...] + p_ref[...]
        x_ref[...] = x
        h = _ln_in(x, s_ref[...], b_ref[...])
        qkv = jnp.dot(h, w_ref[...], preferred_element_type=_F32)
        k_scr[pl.ds(i * TT, TT), :] = _pad_heads(qkv, D_A)
        v_scr[pl.ds(i * TT, TT), :] = _pad_heads(qkv, 2 * D_A)

        @pl.when(i % 2 == 0)
        def _():
            k_scr[pl.ds((i + 1) * TT, TT), :] = jnp.zeros((TT, DAP), _F32)
            v_scr[pl.ds((i + 1) * TT, TT), :] = jnp.zeros((TT, DAP), _F32)

        _attn_inner(i, qkv, k_scr, v_scr, a_ref)

    tile = lambda i: (i, 0)
    full = lambda i: (0, 0)
    return pl.pallas_call(
        body,
        grid=(NT,),
        in_specs=[
            pl.BlockSpec((TT, D_A), tile),
            pl.BlockSpec((TT, D_A), tile),
            pl.BlockSpec((1, D_A), full),
            pl.BlockSpec((1, D_A), full),
            pl.BlockSpec((D_A, 3 * D_A), full),
        ],
        out_specs=[
            pl.BlockSpec((TT, D_A), tile),
            pl.BlockSpec((TT, D_A), tile),
        ],
        out_shape=[jax.ShapeDtypeStruct((T, D_A), _F32)] * 2,
        scratch_shapes=[
            pltpu.VMEM((T, DAP), _F32),
            pltpu.VMEM((T, DAP), _F32),
        ],
    )(g, pos, s1, b1, wqkv)


def _qkvattn_b_call(h_a, au, bv, alpha16, w_base, gamma, bb, asm_s, asm_b,
                    s1, b1, wqkv):
    def body(x_ref, au_ref, bv_ref, al_ref, wb_ref, g_ref, bb_ref, as_ref,
             ab_ref, s_ref, b_ref, w_ref, a_ref, hm_ref, k_scr, v_scr,
             wm_scr):
        i = pl.program_id(0)

        @pl.when(i == 0)
        def _():
            delta = jnp.dot(au_ref[...] * al_ref[...], bv_ref[...],
                            preferred_element_type=_F32)
            wm_scr[...] = wb_ref[...] + g_ref[0] * delta

        t = lax.dot_general(x_ref[...], wm_scr[...],
                            (((1,), (1,)), ((), ())),
                            preferred_element_type=_F32) + bb_ref[...]
        hm = _ln_in(t, as_ref[...], ab_ref[...])
        hm_ref[...] = hm
        h = _ln_in(hm, s_ref[...], b_ref[...])
        qkv = jnp.dot(h, w_ref[...], preferred_element_type=_F32)
        k_scr[pl.ds(i * TT, TT), :] = _pad_heads(qkv, D_A)
        v_scr[pl.ds(i * TT, TT), :] = _pad_heads(qkv, 2 * D_A)

        @pl.when(i % 2 == 0)
        def _():
            k_scr[pl.ds((i + 1) * TT, TT), :] = jnp.zeros((TT, DAP), _F32)
            v_scr[pl.ds((i + 1) * TT, TT), :] = jnp.zeros((TT, DAP), _F32)

        _attn_inner(i, qkv, k_scr, v_scr, a_ref)

    tile = lambda i: (i, 0)
    full = lambda i: (0, 0)
    return pl.pallas_call(
        body,
        grid=(NT,),
        in_specs=[
            pl.BlockSpec((TT, D_A), tile),
            pl.BlockSpec((D_B, 2 * TOP_K), full),
            pl.BlockSpec((2 * TOP_K, D_A), full),
            pl.BlockSpec((1, 2 * TOP_K), full),
            pl.BlockSpec((D_B, D_A), full),
            pl.BlockSpec(memory_space=pltpu.SMEM),
            pl.BlockSpec((1, D_B), full),
            pl.BlockSpec((1, D_B), full),
            pl.BlockSpec((1, D_B), full),
            pl.BlockSpec((1, D_A), full),
            pl.BlockSpec((1, D_A), full),
            pl.BlockSpec((D_A, 3 * D_A), full),
        ],
        out_specs=[
            pl.BlockSpec((TT, D_A), tile),
            pl.BlockSpec((TT, D_B), tile),
        ],
        out_shape=[jax.ShapeDtypeStruct((T, D_A), _F32),
                   jax.ShapeDtypeStruct((T, D_B), _F32)],
        scratch_shapes=[
            pltpu.VMEM((T, DAP), _F32),
            pltpu.VMEM((T, DAP), _F32),
            pltpu.VMEM((D_B, D_A), _F32),
        ],
    )(h_a, au, bv, alpha16, w_base, gamma, bb, asm_s, asm_b, s1, b1, wqkv)


def _woffn_a_call(x, attn, wo, s2, b2, w1, bb1, w2, bb2, wq2, wk2):
    def body(x_ref, a_ref, wo_ref, s_ref, b_ref, w1_ref, b1_ref,
             w2_ref, b2_ref, wq_ref, wk_ref, y_ref, m_ref, z_scr):
        i = pl.program_id(0)
        x1 = x_ref[...] + jnp.dot(a_ref[...], wo_ref[...],
                                  preferred_element_type=_F32)
        h2 = _ln_in(x1, s_ref[...], b_ref[...])
        ff = jax.nn.gelu(jnp.dot(h2, w1_ref[...],
                                 preferred_element_type=_F32) + b1_ref[...])
        y = x1 + jnp.dot(ff, w2_ref[...],
                         preferred_element_type=_F32) + b2_ref[...]
        y_ref[...] = y
        zp = jnp.sum(y, axis=0, keepdims=True)

        @pl.when(i == 0)
        def _():
            z_scr[...] = zp

        @pl.when(i > 0)
        def _():
            z_scr[...] = z_scr[...] + zp

        @pl.when(i == NT - 1)
        def _():
            z = z_scr[...] * (1.0 / T)
            qf = jnp.dot(z, wq_ref[...], preferred_element_type=_F32)
            m = lax.dot_general(qf, wk_ref[...], (((1,), (1,)), ((), ())),
                                preferred_element_type=_F32)
            m_ref[...] = m * 0.125

    tile = lambda i: (i, 0)
    full = lambda i: (0, 0)
    return pl.pallas_call(
        body,
        grid=(NT,),
        in_specs=[
            pl.BlockSpec((TT, D_A), tile),
            pl.BlockSpec((TT, D_A), tile),
            pl.BlockSpec((D_A, D_A), full),
            pl.BlockSpec((1, D_A), full),
            pl.BlockSpec((1, D_A), full),
            pl.BlockSpec((D_A, D_FF), full),
            pl.BlockSpec((1, D_FF), full),
            pl.BlockSpec((D_FF, D_A), full),
            pl.BlockSpec((1, D_A), full),
            pl.BlockSpec((D_A, N_ASPECTS * D_K), full),
            pl.BlockSpec((D_POOL, N_ASPECTS * D_K), full),
        ],
        out_specs=[
            pl.BlockSpec((TT, D_A), tile),
            pl.BlockSpec((1, D_POOL), full),
        ],
        out_shape=[jax.ShapeDtypeStruct((T, D_A), _F32),
                   jax.ShapeDtypeStruct((1, D_POOL), _F32)],
        scratch_shapes=[pltpu.VMEM((1, D_A), _F32)],
    )(x, attn, wo, s2, b2, w1, bb1, w2, bb2, wq2, wk2)


def _woffn_b_call(x, attn, wo, s2, b2, w1, bb1, w2, bb2):
    def body(x_ref, a_ref, wo_ref, s_ref, b_ref, w1_ref, b1_ref,
             w2_ref, b2_ref, y_ref):
        x1 = x_ref[...] + jnp.dot(a_ref[...], wo_ref[...],
                                  preferred_element_type=_F32)
        h2 = _ln_in(x1, s_ref[...], b_ref[...])
        ff = jax.nn.gelu(jnp.dot(h2, w1_ref[...],
                                 preferred_element_type=_F32) + b1_ref[...])
        y_ref[...] = x1 + jnp.dot(ff, w2_ref[...],
                                  preferred_element_type=_F32) + b2_ref[...]

    tile = lambda i: (i, 0)
    full = lambda i: (0, 0)
    return pl.pallas_call(
        body,
        grid=(NT,),
        in_specs=[
            pl.BlockSpec((TT, D_A), tile),
            pl.BlockSpec((TT, D_A), tile),
            pl.BlockSpec((D_A, D_A), full),
            pl.BlockSpec((1, D_A), full),
            pl.BlockSpec((1, D_A), full),
            pl.BlockSpec((D_A, D_FF), full),
            pl.BlockSpec((1, D_FF), full),
            pl.BlockSpec((D_FF, D_A), full),
            pl.BlockSpec((1, D_A), full),
        ],
        out_specs=pl.BlockSpec((TT, D_A), tile),
        out_shape=jax.ShapeDtypeStruct((T, D_A), _F32),
    )(x, attn, wo, s2, b2, w1, bb1, w2, bb2)


def _score_topk_call(pool, m, lam, warm):
    def body(p_ref, m_ref, lam_ref, warm_ref, a_ref, i_ref, c_scr):
        i = pl.program_id(0)
        c_scr[pl.ds(i, 1), :] = lax.dot_general(
            m_ref[...], p_ref[...], (((1,), (1,)), ((), ())),
            preferred_element_type=_F32)

        @pl.when(i == NPT - 1)
        def _():
            c = c_scr[...] * lam_ref[0]
            cmax = jnp.max(c)
            e = jnp.exp(c - cmax)
            soft = e / jnp.sum(e)
            flat = (lax.broadcasted_iota(jnp.int32, (NPT, PT), 0) * PT
                    + lax.broadcasted_iota(jnp.int32, (NPT, PT), 1))
            cur = soft
            vals = []
            for kk in range(TOP_K):
                mx = jnp.max(cur)
                am = jnp.min(jnp.where(cur == mx, flat, jnp.int32(N_POOL)))
                vals.append(mx)
                i_ref[kk] = am
                cur = jnp.where(flat == am, _F32(-1.0), cur)
            vsum = vals[0]
            for kk in range(1, TOP_K):
                vsum = vsum + vals[kk]
            warmb = warm_ref[0] != 0
            for kk in range(TOP_K):
                a_ref[kk] = jnp.where(warmb, vals[kk],
                                      vals[kk] / (vsum + 1e-9))

    return pl.pallas_call(
        body,
        grid=(NPT,),
        in_specs=[
            pl.BlockSpec((PT, D_POOL), lambda i: (i, 0)),
            pl.BlockSpec((1, D_POOL), lambda i: (0, 0)),
            pl.BlockSpec(memory_space=pltpu.SMEM),
            pl.BlockSpec(memory_space=pltpu.SMEM),
        ],
        out_specs=[
            pl.BlockSpec(memory_space=pltpu.SMEM),
            pl.BlockSpec(memory_space=pltpu.SMEM),
        ],
        out_shape=[
            jax.ShapeDtypeStruct((TOP_K,), _F32),
            jax.ShapeDtypeStruct((TOP_K,), jnp.int32),
        ],
        scratch_shapes=[pltpu.VMEM((NPT, PT), _F32)],
    )(pool, m, lam, warm)


def _lmhead_call(x, w):
    def body(x_ref, w_ref, o_ref):
        o_ref[...] = jnp.dot(x_ref[...], w_ref[...],
                             preferred_element_type=_F32)

    return pl.pallas_call(
        body,
        grid=(NVT,),
        in_specs=[
            pl.BlockSpec((T, D_B), lambda j: (0, 0)),
            pl.BlockSpec((D_B, VT), lambda j: (0, j)),
        ],
        out_specs=pl.BlockSpec((T, VT), lambda j: (0, j)),
        out_shape=jax.ShapeDtypeStruct((T, VOCAB), _F32),
    )(x, w)


def kernel(input_ids, lambda_val, is_warmup, embed_table, a_ln1_s, a_ln1_b,
           a_wqkv, a_wo, a_ln2_s, a_ln2_b, a_w1, a_b1, a_w2, a_b2,
           pool_vectors, w_key, w_query, w_base, b_base, gamma, asm_ln_s,
           asm_ln_b, b_ln1_s, b_ln1_b, b_wqkv, b_wo, b_ln2_s, b_ln2_b,
           b_w1, b_b1, b_w2, b_b2, lm_head_w):
    row2 = lambda a: jnp.asarray(a, _F32).reshape(1, -1)

    ids = input_ids.reshape(T).astype(jnp.int32)
    g = _embed_gather(embed_table, ids)
    pos = jnp.asarray(_POS_ENC)

    attn, x = _qkvattn_a_call(g, pos, row2(a_ln1_s), row2(a_ln1_b), a_wqkv)
    wq2 = w_query.transpose(1, 0, 2).reshape(D_A, N_ASPECTS * D_K)
    wk2 = w_key.transpose(1, 0, 2).reshape(D_POOL, N_ASPECTS * D_K)
    h_a, m = _woffn_a_call(x, attn, a_wo, row2(a_ln2_s), row2(a_ln2_b),
                           a_w1, row2(a_b1), a_w2, row2(a_b2), wq2, wk2)

    lam = jnp.asarray(lambda_val, _F32).reshape(1)
    warm = jnp.asarray(is_warmup, jnp.int32).reshape(1)
    alphas, indices = _score_topk_call(pool_vectors, m, lam, warm)

    gathered = _pool_gather(pool_vectors, indices)
    au = gathered[:, :D_B * R].reshape(TOP_K, D_B, R).transpose(1, 0, 2)
    au = au.reshape(D_B, TOP_K * R)
    bv = gathered[:, D_B * R:].reshape(TOP_K * R, D_A)
    alpha16 = jnp.repeat(alphas, R).reshape(1, TOP_K * R)
    attn2, h_mid = _qkvattn_b_call(h_a, au, bv, alpha16, w_base,
                                   gamma.reshape(1), row2(b_base),
                                   row2(asm_ln_s), row2(asm_ln_b),
                                   row2(b_ln1_s), row2(b_ln1_b), b_wqkv)
    h_out = _woffn_b_call(h_mid, attn2, b_wo, row2(b_ln2_s), row2(b_ln2_b),
                          b_w1, row2(b_b1), b_w2, row2(b_b2))

    logits = _lmhead_call(h_out, lm_head_w)
    return logits.reshape(1, T, VOCAB)
